# baseline (speedup 1.0000x reference)
import jax
from jax.experimental import pallas as pl
from jax.experimental.pallas import tpu as pltpu

_CHUNK_ROWS = 256


def _copy_body(pe_ref, out_ref, buf, in_sems, out_sems):
    seq_len = out_ref.shape[1]
    n_chunks = seq_len // _CHUNK_ROWS

    def cp_in(i):
        return pltpu.make_async_copy(
            pe_ref.at[:, pl.ds(i * _CHUNK_ROWS, _CHUNK_ROWS), :],
            buf.at[i],
            in_sems.at[i],
        )

    def cp_out(i):
        return pltpu.make_async_copy(
            buf.at[i],
            out_ref.at[:, pl.ds(i * _CHUNK_ROWS, _CHUNK_ROWS), :],
            out_sems.at[i],
        )

    for i in range(n_chunks):
        cp_in(i).start()
    for i in range(n_chunks):
        cp_in(i).wait()
        cp_out(i).start()
    for i in range(n_chunks):
        cp_out(i).wait()


def kernel(x, pe):
    seq_len = x.shape[1]
    d_model = pe.shape[2]
    n_chunks = seq_len // _CHUNK_ROWS
    out_shape = jax.ShapeDtypeStruct((1, seq_len, d_model), pe.dtype)
    return pl.pallas_call(
        _copy_body,
        out_shape=out_shape,
        in_specs=[pl.BlockSpec(memory_space=pl.ANY)],
        out_specs=pl.BlockSpec(memory_space=pl.ANY),
        scratch_shapes=[
            pltpu.VMEM((n_chunks, 1, _CHUNK_ROWS, d_model), pe.dtype),
            pltpu.SemaphoreType.DMA((n_chunks,)),
            pltpu.SemaphoreType.DMA((n_chunks,)),
        ],
    )(pe)


# speedup vs baseline: 1.1247x; 1.1247x over previous
import jax
from jax.experimental import pallas as pl
from jax.experimental.pallas import tpu as pltpu

_CHUNK_ROWS = 1024


def _copy_body(pe_ref, out_ref, buf, in_sems, out_sems):
    seq_len = out_ref.shape[1]
    n_chunks = seq_len // _CHUNK_ROWS

    def cp_in(i):
        return pltpu.make_async_copy(
            pe_ref.at[:, pl.ds(i * _CHUNK_ROWS, _CHUNK_ROWS), :],
            buf.at[i],
            in_sems.at[i],
        )

    def cp_out(i):
        return pltpu.make_async_copy(
            buf.at[i],
            out_ref.at[:, pl.ds(i * _CHUNK_ROWS, _CHUNK_ROWS), :],
            out_sems.at[i],
        )

    for i in range(n_chunks):
        cp_in(i).start()
    for i in range(n_chunks):
        cp_in(i).wait()
        cp_out(i).start()
    for i in range(n_chunks):
        cp_out(i).wait()


def kernel(x, pe):
    seq_len = x.shape[1]
    d_model = pe.shape[2]
    n_chunks = seq_len // _CHUNK_ROWS
    out_shape = jax.ShapeDtypeStruct((1, seq_len, d_model), pe.dtype)
    return pl.pallas_call(
        _copy_body,
        out_shape=out_shape,
        in_specs=[pl.BlockSpec(memory_space=pl.ANY)],
        out_specs=pl.BlockSpec(memory_space=pl.ANY),
        scratch_shapes=[
            pltpu.VMEM((n_chunks, 1, _CHUNK_ROWS, d_model), pe.dtype),
            pltpu.SemaphoreType.DMA((n_chunks,)),
            pltpu.SemaphoreType.DMA((n_chunks,)),
        ],
    )(pe)
